# baseline (speedup 1.0000x reference)
import functools

import jax
import jax.numpy as jnp
from jax import lax
from jax.experimental import pallas as pl
from jax.experimental.pallas import tpu as pltpu
from jax.experimental.pallas import tpu_sc as plsc

N = 10000
NP = 10240
E = 320000
NC, NS, L = 2, 16, 16
RPT = NP // NS
SEG = 2048
EP = 2 * 80 * SEG
HE = EP // NC
NSEG = HE // SEG
SEGV = SEG // L
FLUSH = 1024
STCAP = 1184
CAPB = HE + FLUSH
BK = 128

_SC_PARAMS = pltpu.CompilerParams(
    use_tc_tiling_on_sc=False, needs_layout_passes=False)


@functools.cache
def _mesh():
    return plsc.VectorSubcoreMesh(
        core_axis_name="c", subcore_axis_name="s",
        num_cores=NC, num_subcores=NS,
    )


def _part_body(src_hbm, dst_hbm, attr_hbm, pk_hbm, ab_hbm, cnt_hbm,
               si0, di0, ai0, si1, di1, ai1, pkst, atst, cntv, sm0, sm1):
    c = lax.axis_index("c")
    t = lax.axis_index("s")
    base = c * HE
    sin = (si0, si1)
    din = (di0, di1)
    ain = (ai0, ai1)
    sems = (sm0, sm1)
    ones = jnp.full((L,), 1, jnp.int32)
    zeros = jnp.full((L,), 0, jnp.int32)

    pltpu.async_copy(src_hbm.at[pl.ds(base, SEG)], si0, sm0)
    pltpu.async_copy(dst_hbm.at[pl.ds(base, SEG)], di0, sm0)
    pltpu.async_copy(attr_hbm.at[pl.ds(base, SEG)], ai0, sm0)

    def _seg_group(gr, carry):
        sp, gc = carry
        for b in range(2):
            seg = gr * 2 + b
            off = base + seg * SEG
            pltpu.make_async_copy(
                src_hbm.at[pl.ds(off, SEG)], sin[b], sems[b]).wait()
            pltpu.make_async_copy(
                dst_hbm.at[pl.ds(off, SEG)], din[b], sems[b]).wait()
            pltpu.make_async_copy(
                attr_hbm.at[pl.ds(off, SEG)], ain[b], sems[b]).wait()

            nxt = 1 - b

            @pl.when(seg + 1 < NSEG)
            def _():
                noff = base + (seg + 1) * SEG
                pltpu.async_copy(
                    src_hbm.at[pl.ds(noff, SEG)], sin[nxt], sems[nxt])
                pltpu.async_copy(
                    dst_hbm.at[pl.ds(noff, SEG)], din[nxt], sems[nxt])
                pltpu.async_copy(
                    attr_hbm.at[pl.ds(noff, SEG)], ain[nxt], sems[nxt])

            def _vloop(j, carry2, b=b):
                sp, gc = carry2
                d = din[b][pl.ds(j * L, L)]
                sv = sin[b][pl.ds(j * L, L)]
                av = ain[b][pl.ds(j * L, L)]
                bucket = lax.shift_right_arithmetic(d * 6554, 22)
                m = bucket == t
                mi = jnp.where(m, ones, zeros)
                cs = plsc.cumsum(mi)
                pos = sp + cs - 1
                packed = sv | lax.shift_left(d - t * RPT, 14)
                plsc.store_scatter(pkst, [pos], packed, mask=m)
                plsc.store_scatter(atst, [pos], av, mask=m)
                sp = sp + cs[15]
                fl = sp >= FLUSH

                @pl.when(fl)
                def _():
                    gca = pl.multiple_of(gc, FLUSH)
                    pltpu.sync_copy(pkst.at[pl.ds(0, FLUSH)],
                                    pk_hbm.at[c, t, pl.ds(gca, FLUSH)])
                    pltpu.sync_copy(atst.at[pl.ds(0, FLUSH)],
                                    ab_hbm.at[c, t, pl.ds(gca, FLUSH)])
                    pkst[pl.ds(0, L)] = pkst[pl.ds(FLUSH, L)]
                    atst[pl.ds(0, L)] = atst[pl.ds(FLUSH, L)]

                sp = jnp.where(fl, sp - FLUSH, sp)
                gc = jnp.where(fl, gc + FLUSH, gc)
                return sp, gc

            sp, gc = lax.fori_loop(0, SEGV, _vloop, (sp, gc))
        return sp, gc

    sp, gc = lax.fori_loop(0, NSEG // 2, _seg_group,
                           (jnp.int32(0), jnp.int32(0)))

    iota = lax.broadcasted_iota(jnp.int32, (L,), 0)
    zi = jnp.zeros((L,), jnp.int32)
    zf = jnp.zeros((L,), jnp.float32)

    def _zfill(i, _):
        pos = sp + i * L + iota
        m = pos < FLUSH
        plsc.store_scatter(pkst, [pos], zi, mask=m)
        plsc.store_scatter(atst, [pos], zf, mask=m)
        return 0
    lax.fori_loop(0, FLUSH // L, _zfill, 0)
    gca = pl.multiple_of(gc, FLUSH)
    pltpu.sync_copy(pkst.at[pl.ds(0, FLUSH)],
                    pk_hbm.at[c, t, pl.ds(gca, FLUSH)])
    pltpu.sync_copy(atst.at[pl.ds(0, FLUSH)],
                    ab_hbm.at[c, t, pl.ds(gca, FLUSH)])
    gc = gc + FLUSH
    cntv[pl.ds(0, L)] = jnp.full((L,), gc, jnp.int32)
    pltpu.sync_copy(cntv, cnt_hbm.at[c, t])


@functools.cache
def _make_partition():
    return functools.partial(
        pl.kernel,
        out_type=(
            jax.ShapeDtypeStruct((NC, NS, CAPB), jnp.int32),
            jax.ShapeDtypeStruct((NC, NS, CAPB), jnp.float32),
            jax.ShapeDtypeStruct((NC, NS, L), jnp.int32),
        ),
        mesh=_mesh(),
        scratch_types=[
            pltpu.VMEM((SEG,), jnp.int32),
            pltpu.VMEM((SEG,), jnp.int32),
            pltpu.VMEM((SEG,), jnp.float32),
            pltpu.VMEM((SEG,), jnp.int32),
            pltpu.VMEM((SEG,), jnp.int32),
            pltpu.VMEM((SEG,), jnp.float32),
            pltpu.VMEM((STCAP,), jnp.int32),
            pltpu.VMEM((STCAP,), jnp.float32),
            pltpu.VMEM((L,), jnp.int32),
            pltpu.SemaphoreType.DMA,
            pltpu.SemaphoreType.DMA,
        ],
        compiler_params=_SC_PARAMS,
    )(_part_body)


def _agg_body(fo, h_hbm, pk_hbm, ab_hbm, cnt_hbm, out_hbm,
              pk0, pk1, ab0, ab1, ix0, ix1, rb0, rb1, acc, cntv,
              p0, p1, g0, g1):
    pkb = (pk0, pk1)
    abb = (ab0, ab1)
    ixb = (ix0, ix1)
    rbb = (rb0, rb1)
    psem = (p0, p1)
    gsem = (g0, g1)
    c = lax.axis_index("c")
    t = lax.axis_index("s")
    mask14 = jnp.full((L,), 0x3FFF, jnp.int32)

    pltpu.sync_copy(cnt_hbm.at[c, t], cntv)
    nent = cntv[pl.ds(0, L)][0]
    nb = lax.shift_right_logical(nent, 7)

    def _z(r, _):
        for q in range(fo // L):
            acc[r, pl.ds(q * L, L)] = jnp.zeros((L,), jnp.float32)
        return 0
    lax.fori_loop(0, RPT, _z, 0)

    pltpu.async_copy(pk_hbm.at[c, t, pl.ds(0, BK)], pk0, p0)
    pltpu.async_copy(ab_hbm.at[c, t, pl.ds(0, BK)], ab0, p0)

    def _slot(i, b):
        @pl.when(i < nb)
        def _():
            pltpu.make_async_copy(
                pk_hbm.at[c, t, pl.ds(i * BK, BK)], pkb[b], psem[b]).wait()
            pltpu.make_async_copy(
                ab_hbm.at[c, t, pl.ds(i * BK, BK)], abb[b], psem[b]).wait()
            for j in range(BK // L):
                sl = pl.ds(j * L, L)
                ixb[b][sl] = pkb[b][sl] & mask14
            pltpu.async_copy(h_hbm.at[ixb[b]], rbb[b], gsem[b])

        @pl.when(jnp.logical_and(i >= 1, i <= nb))
        def _():
            ob = 1 - b
            pltpu.make_async_copy(
                h_hbm.at[ixb[ob]], rbb[ob], gsem[ob]).wait()

            def _acc(j, _):
                sl = pl.ds(j * L, L)
                pkv = pkb[ob][sl]
                dstl = lax.shift_right_arithmetic(pkv, 14)
                av = abb[ob][sl]
                for tt in range(L):
                    e = j * L + tt
                    row = dstl[tt]
                    sp = jnp.full((L,), av[tt], jnp.float32)
                    for q in range(fo // L):
                        qs = pl.ds(q * L, L)
                        plsc.addupdate(acc.at[row, qs], rbb[ob][e, qs] * sp)
                return 0
            lax.fori_loop(0, BK // L, _acc, 0)

        @pl.when(i + 1 < nb)
        def _():
            ob = 1 - b
            pltpu.async_copy(
                pk_hbm.at[c, t, pl.ds((i + 1) * BK, BK)], pkb[ob], psem[ob])
            pltpu.async_copy(
                ab_hbm.at[c, t, pl.ds((i + 1) * BK, BK)], abb[ob], psem[ob])

    def _group(gi, _):
        for b in range(2):
            _slot(gi * 2 + b, b)
        return 0
    ngr = lax.shift_right_logical(nb + 2, 1)
    lax.fori_loop(0, ngr, _group, 0)

    pltpu.sync_copy(acc, out_hbm.at[c, pl.ds(t * RPT, RPT)])


@functools.cache
def _make_agg(fo):
    return functools.partial(
        pl.kernel,
        out_type=jax.ShapeDtypeStruct((NC, NP, fo), jnp.float32),
        mesh=_mesh(),
        scratch_types=[
            pltpu.VMEM((BK,), jnp.int32),
            pltpu.VMEM((BK,), jnp.int32),
            pltpu.VMEM((BK,), jnp.float32),
            pltpu.VMEM((BK,), jnp.float32),
            pltpu.VMEM((BK,), jnp.int32),
            pltpu.VMEM((BK,), jnp.int32),
            pltpu.VMEM((BK, fo), jnp.float32),
            pltpu.VMEM((BK, fo), jnp.float32),
            pltpu.VMEM((RPT, fo), jnp.float32),
            pltpu.VMEM((L,), jnp.int32),
            pltpu.SemaphoreType.DMA,
            pltpu.SemaphoreType.DMA,
            pltpu.SemaphoreType.DMA,
            pltpu.SemaphoreType.DMA,
        ],
        compiler_params=_SC_PARAMS,
    )(functools.partial(_agg_body, fo))


_BR = 1024


def _mm1_body(x_ref, w_ref, o_ref):
    o_ref[...] = jnp.dot(x_ref[...], w_ref[...],
                         preferred_element_type=jnp.float32)


def _mm1(x, W):
    fi, fo = W.shape
    return pl.pallas_call(
        _mm1_body,
        grid=(NP // _BR,),
        in_specs=[pl.BlockSpec((_BR, fi), lambda i: (i, 0)),
                  pl.BlockSpec((fi, fo), lambda i: (0, 0))],
        out_specs=pl.BlockSpec((_BR, fo), lambda i: (i, 0)),
        out_shape=jax.ShapeDtypeStruct((NP, fo), jnp.float32),
    )(x, W)


def _fused_body(p_ref, b_ref, w_ref, o_ref):
    h = jnp.maximum(p_ref[0] + p_ref[1] + b_ref[...], 0.0)
    o_ref[...] = jnp.dot(h, w_ref[...], preferred_element_type=jnp.float32)


def _fused(p, b, W):
    fi, fo = W.shape
    return pl.pallas_call(
        _fused_body,
        grid=(NP // _BR,),
        in_specs=[pl.BlockSpec((2, _BR, fi), lambda i: (0, i, 0)),
                  pl.BlockSpec((1, fi), lambda i: (0, 0)),
                  pl.BlockSpec((fi, fo), lambda i: (0, 0))],
        out_specs=pl.BlockSpec((_BR, fo), lambda i: (i, 0)),
        out_shape=jax.ShapeDtypeStruct((NP, fo), jnp.float32),
    )(p, b.reshape(1, fi), W)


def _final_body(p_ref, b_ref, o_ref):
    z = p_ref[0, :, 0:1] + p_ref[1, :, 0:1] + b_ref[0, 0]
    o_ref[...] = jax.nn.sigmoid(z)


def _final(p, b):
    return pl.pallas_call(
        _final_body,
        grid=(NP // _BR,),
        in_specs=[pl.BlockSpec((2, _BR, 16), lambda i: (0, i, 0)),
                  pl.BlockSpec((1, 1), lambda i: (0, 0))],
        out_specs=pl.BlockSpec((_BR, 1), lambda i: (i, 0)),
        out_shape=jax.ShapeDtypeStruct((NP, 1), jnp.float32),
    )(p, b.reshape(1, 1))


def kernel(x, edge_index, edge_attr, W1, b1, W2, b2, W3, b3, W4, b4):
    x_p = jnp.zeros((NP, x.shape[1]), jnp.float32).at[:N].set(x)
    pad = EP - E
    src_p = jnp.concatenate([edge_index[0], jnp.zeros((pad,), jnp.int32)])
    dst_p = jnp.concatenate([edge_index[1], jnp.zeros((pad,), jnp.int32)])
    attr_p = jnp.concatenate([edge_attr, jnp.zeros((pad,), jnp.float32)])
    W4p = jnp.zeros((16, 16), jnp.float32).at[:, 0:1].set(W4)

    pk, ab, cnt = _make_partition()(src_p, dst_p, attr_p)

    h1 = _mm1(x_p, W1)
    p1 = _make_agg(64)(h1, pk, ab, cnt)
    h2 = _fused(p1, b1, W2)
    p2 = _make_agg(32)(h2, pk, ab, cnt)
    h3 = _fused(p2, b2, W3)
    p3 = _make_agg(16)(h3, pk, ab, cnt)
    h4 = _fused(p3, b3, W4p)
    p4 = _make_agg(16)(h4, pk, ab, cnt)
    out = _final(p4, b4)
    return out[:N]


# speedup vs baseline: 1.0668x; 1.0668x over previous
import functools

import jax
import jax.numpy as jnp
from jax import lax
from jax.experimental import pallas as pl
from jax.experimental.pallas import tpu as pltpu
from jax.experimental.pallas import tpu_sc as plsc

N = 10000
NP = 10240
E = 320000
NC, NS, L = 2, 16, 16
RPT = NP // NS
SEG = 2048
EP = 2 * 80 * SEG
HE = EP // NC
NSEG = HE // SEG
SEGV = SEG // L
FLUSH = 1024
STCAP = 1184
CAPB = HE + FLUSH
BK = 128

_SC_PARAMS = pltpu.CompilerParams(
    use_tc_tiling_on_sc=False, needs_layout_passes=False)


@functools.cache
def _mesh():
    return plsc.VectorSubcoreMesh(
        core_axis_name="c", subcore_axis_name="s",
        num_cores=NC, num_subcores=NS,
    )


def _part_body(src_hbm, dst_hbm, attr_hbm, pk_hbm, ab_hbm, cnt_hbm,
               si0, di0, ai0, si1, di1, ai1, pkst, atst, cntv, sm0, sm1):
    c = lax.axis_index("c")
    t = lax.axis_index("s")
    base = c * HE
    sin = (si0, si1)
    din = (di0, di1)
    ain = (ai0, ai1)
    sems = (sm0, sm1)
    ones = jnp.full((L,), 1, jnp.int32)
    zeros = jnp.full((L,), 0, jnp.int32)

    pltpu.async_copy(src_hbm.at[pl.ds(base, SEG)], si0, sm0)
    pltpu.async_copy(dst_hbm.at[pl.ds(base, SEG)], di0, sm0)
    pltpu.async_copy(attr_hbm.at[pl.ds(base, SEG)], ai0, sm0)

    def _seg_group(gr, carry):
        spv, gc = carry
        for b in range(2):
            seg = gr * 2 + b
            off = base + seg * SEG
            pltpu.make_async_copy(
                src_hbm.at[pl.ds(off, SEG)], sin[b], sems[b]).wait()
            pltpu.make_async_copy(
                dst_hbm.at[pl.ds(off, SEG)], din[b], sems[b]).wait()
            pltpu.make_async_copy(
                attr_hbm.at[pl.ds(off, SEG)], ain[b], sems[b]).wait()

            nxt = 1 - b

            @pl.when(seg + 1 < NSEG)
            def _():
                noff = base + (seg + 1) * SEG
                pltpu.async_copy(
                    src_hbm.at[pl.ds(noff, SEG)], sin[nxt], sems[nxt])
                pltpu.async_copy(
                    dst_hbm.at[pl.ds(noff, SEG)], din[nxt], sems[nxt])
                pltpu.async_copy(
                    attr_hbm.at[pl.ds(noff, SEG)], ain[nxt], sems[nxt])

            def _vgroup(j4, carry2, b=b):
                spv, gc = carry2
                for u in range(4):
                    j = j4 * 4 + u
                    d = din[b][pl.ds(j * L, L)]
                    sv = sin[b][pl.ds(j * L, L)]
                    av = ain[b][pl.ds(j * L, L)]
                    bucket = lax.shift_right_arithmetic(d * 6554, 22)
                    m = bucket == t
                    mi = jnp.where(m, ones, zeros)
                    cs = plsc.cumsum(mi)
                    tot = spv + cs
                    pos = tot - 1
                    packed = sv | lax.shift_left(d - t * RPT, 14)
                    plsc.store_scatter(pkst, [pos], packed, mask=m)
                    plsc.store_scatter(atst, [pos], av, mask=m)
                    spv = jnp.full((L,), tot[15], jnp.int32)
                fl = spv[0] >= FLUSH

                @pl.when(fl)
                def _():
                    gca = pl.multiple_of(gc, FLUSH)
                    pltpu.sync_copy(pkst.at[pl.ds(0, FLUSH)],
                                    pk_hbm.at[c, t, pl.ds(gca, FLUSH)])
                    pltpu.sync_copy(atst.at[pl.ds(0, FLUSH)],
                                    ab_hbm.at[c, t, pl.ds(gca, FLUSH)])
                    for k in range(5):
                        ks = pl.ds(k * L, L)
                        kf = pl.ds(FLUSH + k * L, L)
                        pkst[ks] = pkst[kf]
                        atst[ks] = atst[kf]

                spv = jnp.where(fl, spv - FLUSH, spv)
                gc = jnp.where(fl, gc + FLUSH, gc)
                return spv, gc

            spv, gc = lax.fori_loop(0, SEGV // 4, _vgroup, (spv, gc))
        return spv, gc

    spv, gc = lax.fori_loop(0, NSEG // 2, _seg_group,
                            (jnp.full((L,), 0, jnp.int32), jnp.int32(0)))
    sp = spv[0]

    iota = lax.broadcasted_iota(jnp.int32, (L,), 0)
    zi = jnp.zeros((L,), jnp.int32)
    zf = jnp.zeros((L,), jnp.float32)

    def _zfill(i, _):
        pos = sp + i * L + iota
        m = pos < FLUSH
        plsc.store_scatter(pkst, [pos], zi, mask=m)
        plsc.store_scatter(atst, [pos], zf, mask=m)
        return 0
    lax.fori_loop(0, FLUSH // L, _zfill, 0)
    gca = pl.multiple_of(gc, FLUSH)
    pltpu.sync_copy(pkst.at[pl.ds(0, FLUSH)],
                    pk_hbm.at[c, t, pl.ds(gca, FLUSH)])
    pltpu.sync_copy(atst.at[pl.ds(0, FLUSH)],
                    ab_hbm.at[c, t, pl.ds(gca, FLUSH)])
    gc = gc + FLUSH
    cntv[pl.ds(0, L)] = jnp.full((L,), gc, jnp.int32)
    pltpu.sync_copy(cntv, cnt_hbm.at[c, t])


@functools.cache
def _make_partition():
    return functools.partial(
        pl.kernel,
        out_type=(
            jax.ShapeDtypeStruct((NC, NS, CAPB), jnp.int32),
            jax.ShapeDtypeStruct((NC, NS, CAPB), jnp.float32),
            jax.ShapeDtypeStruct((NC, NS, L), jnp.int32),
        ),
        mesh=_mesh(),
        scratch_types=[
            pltpu.VMEM((SEG,), jnp.int32),
            pltpu.VMEM((SEG,), jnp.int32),
            pltpu.VMEM((SEG,), jnp.float32),
            pltpu.VMEM((SEG,), jnp.int32),
            pltpu.VMEM((SEG,), jnp.int32),
            pltpu.VMEM((SEG,), jnp.float32),
            pltpu.VMEM((STCAP,), jnp.int32),
            pltpu.VMEM((STCAP,), jnp.float32),
            pltpu.VMEM((L,), jnp.int32),
            pltpu.SemaphoreType.DMA,
            pltpu.SemaphoreType.DMA,
        ],
        compiler_params=_SC_PARAMS,
    )(_part_body)


def _agg_body(fo, h_hbm, pk_hbm, ab_hbm, cnt_hbm, out_hbm,
              pk0, pk1, ab0, ab1, ix0, ix1, rb0, rb1, acc, cntv,
              p0, p1, g0, g1):
    pkb = (pk0, pk1)
    abb = (ab0, ab1)
    ixb = (ix0, ix1)
    rbb = (rb0, rb1)
    psem = (p0, p1)
    gsem = (g0, g1)
    c = lax.axis_index("c")
    t = lax.axis_index("s")
    mask14 = jnp.full((L,), 0x3FFF, jnp.int32)

    pltpu.sync_copy(cnt_hbm.at[c, t], cntv)
    nent = cntv[pl.ds(0, L)][0]
    nb = lax.shift_right_logical(nent, 7)

    zv = jnp.zeros((L,), jnp.float32)

    def _z(r, _):
        for u in range(4):
            acc[pl.ds((r * 4 + u) * L, L)] = zv
        return 0
    lax.fori_loop(0, RPT * fo // (4 * L), _z, 0)

    pltpu.async_copy(pk_hbm.at[c, t, pl.ds(0, BK)], pk0, p0)
    pltpu.async_copy(ab_hbm.at[c, t, pl.ds(0, BK)], ab0, p0)

    def _slot(i, b):
        @pl.when(i < nb)
        def _():
            pltpu.make_async_copy(
                pk_hbm.at[c, t, pl.ds(i * BK, BK)], pkb[b], psem[b]).wait()
            pltpu.make_async_copy(
                ab_hbm.at[c, t, pl.ds(i * BK, BK)], abb[b], psem[b]).wait()
            for j in range(BK // L):
                sl = pl.ds(j * L, L)
                ixb[b][sl] = pkb[b][sl] & mask14
            pltpu.async_copy(h_hbm.at[ixb[b]], rbb[b], gsem[b])

        @pl.when(jnp.logical_and(i >= 1, i <= nb))
        def _():
            ob = 1 - b
            pltpu.make_async_copy(
                h_hbm.at[ixb[ob]], rbb[ob], gsem[ob]).wait()

            iota = lax.broadcasted_iota(jnp.int32, (L,), 0)

            def _acc(j, _):
                sl = pl.ds(j * L, L)
                pkv = pkb[ob][sl]
                rowbase = lax.shift_right_arithmetic(pkv, 14) * fo
                av = abb[ob][sl]
                for h in range(2):
                    sps = [jnp.full((L,), av[h * 8 + tt], jnp.float32)
                           for tt in range(8)]
                    rbs = [jnp.full((L,), rowbase[h * 8 + tt], jnp.int32)
                           + iota for tt in range(8)]
                    for q in range(fo // L):
                        for tt in range(8):
                            e = j * L + h * 8 + tt
                            idx = rbs[tt] + (q * L)
                            val = rbb[ob][e, pl.ds(q * L, L)] * sps[tt]
                            plsc.addupdate_scatter(acc, [idx], val)
                return 0
            lax.fori_loop(0, BK // L, _acc, 0)

        @pl.when(i + 1 < nb)
        def _():
            ob = 1 - b
            pltpu.async_copy(
                pk_hbm.at[c, t, pl.ds((i + 1) * BK, BK)], pkb[ob], psem[ob])
            pltpu.async_copy(
                ab_hbm.at[c, t, pl.ds((i + 1) * BK, BK)], abb[ob], psem[ob])

    def _group(gi, _):
        for b in range(2):
            _slot(gi * 2 + b, b)
        return 0
    ngr = lax.shift_right_logical(nb + 2, 1)
    lax.fori_loop(0, ngr, _group, 0)

    pltpu.sync_copy(acc, out_hbm.at[c, pl.ds(t * (RPT * fo), RPT * fo)])


@functools.cache
def _make_agg(fo):
    return functools.partial(
        pl.kernel,
        out_type=jax.ShapeDtypeStruct((NC, NP * fo), jnp.float32),
        mesh=_mesh(),
        scratch_types=[
            pltpu.VMEM((BK,), jnp.int32),
            pltpu.VMEM((BK,), jnp.int32),
            pltpu.VMEM((BK,), jnp.float32),
            pltpu.VMEM((BK,), jnp.float32),
            pltpu.VMEM((BK,), jnp.int32),
            pltpu.VMEM((BK,), jnp.int32),
            pltpu.VMEM((BK, fo), jnp.float32),
            pltpu.VMEM((BK, fo), jnp.float32),
            pltpu.VMEM((RPT * fo,), jnp.float32),
            pltpu.VMEM((L,), jnp.int32),
            pltpu.SemaphoreType.DMA,
            pltpu.SemaphoreType.DMA,
            pltpu.SemaphoreType.DMA,
            pltpu.SemaphoreType.DMA,
        ],
        compiler_params=_SC_PARAMS,
    )(functools.partial(_agg_body, fo))


_BR = 1024


def _mm1_body(x_ref, w_ref, o_ref):
    o_ref[...] = jnp.dot(x_ref[...], w_ref[...],
                         preferred_element_type=jnp.float32)


def _mm1(x, W):
    fi, fo = W.shape
    return pl.pallas_call(
        _mm1_body,
        grid=(NP // _BR,),
        in_specs=[pl.BlockSpec((_BR, fi), lambda i: (i, 0)),
                  pl.BlockSpec((fi, fo), lambda i: (0, 0))],
        out_specs=pl.BlockSpec((_BR, fo), lambda i: (i, 0)),
        out_shape=jax.ShapeDtypeStruct((NP, fo), jnp.float32),
    )(x, W)


def _fused_body(p_ref, b_ref, w_ref, o_ref):
    h = jnp.maximum(p_ref[0] + p_ref[1] + b_ref[...], 0.0)
    o_ref[...] = jnp.dot(h, w_ref[...], preferred_element_type=jnp.float32)


def _fused(p, b, W):
    fi, fo = W.shape
    return pl.pallas_call(
        _fused_body,
        grid=(NP // _BR,),
        in_specs=[pl.BlockSpec((2, _BR, fi), lambda i: (0, i, 0)),
                  pl.BlockSpec((1, fi), lambda i: (0, 0)),
                  pl.BlockSpec((fi, fo), lambda i: (0, 0))],
        out_specs=pl.BlockSpec((_BR, fo), lambda i: (i, 0)),
        out_shape=jax.ShapeDtypeStruct((NP, fo), jnp.float32),
    )(p, b.reshape(1, fi), W)


def _final_body(p_ref, b_ref, o_ref):
    z = p_ref[0, :, 0:1] + p_ref[1, :, 0:1] + b_ref[0, 0]
    o_ref[...] = jax.nn.sigmoid(z)


def _final(p, b):
    return pl.pallas_call(
        _final_body,
        grid=(NP // _BR,),
        in_specs=[pl.BlockSpec((2, _BR, 16), lambda i: (0, i, 0)),
                  pl.BlockSpec((1, 1), lambda i: (0, 0))],
        out_specs=pl.BlockSpec((_BR, 1), lambda i: (i, 0)),
        out_shape=jax.ShapeDtypeStruct((NP, 1), jnp.float32),
    )(p, b.reshape(1, 1))


def kernel(x, edge_index, edge_attr, W1, b1, W2, b2, W3, b3, W4, b4):
    x_p = jnp.zeros((NP, x.shape[1]), jnp.float32).at[:N].set(x)
    pad = EP - E
    src_p = jnp.concatenate([edge_index[0], jnp.zeros((pad,), jnp.int32)])
    dst_p = jnp.concatenate([edge_index[1], jnp.zeros((pad,), jnp.int32)])
    attr_p = jnp.concatenate([edge_attr, jnp.zeros((pad,), jnp.float32)])
    W4p = jnp.zeros((16, 16), jnp.float32).at[:, 0:1].set(W4)

    pk, ab, cnt = _make_partition()(src_p, dst_p, attr_p)

    h1 = _mm1(x_p, W1)
    p1 = _make_agg(64)(h1, pk, ab, cnt).reshape(NC, NP, 64)
    h2 = _fused(p1, b1, W2)
    p2 = _make_agg(32)(h2, pk, ab, cnt).reshape(NC, NP, 32)
    h3 = _fused(p2, b2, W3)
    p3 = _make_agg(16)(h3, pk, ab, cnt).reshape(NC, NP, 16)
    h4 = _fused(p3, b3, W4p)
    p4 = _make_agg(16)(h4, pk, ab, cnt).reshape(NC, NP, 16)
    out = _final(p4, b4)
    return out[:N]


# speedup vs baseline: 1.0807x; 1.0129x over previous
import functools

import jax
import jax.numpy as jnp
from jax import lax
from jax.experimental import pallas as pl
from jax.experimental.pallas import tpu as pltpu
from jax.experimental.pallas import tpu_sc as plsc

N = 10000
NP = 10240
E = 320000
NC, NS, L = 2, 16, 16
RPT = NP // NS
SEG = 2048
EP = 2 * 80 * SEG
HE = EP // NC
NSEG = HE // SEG
SEGV = SEG // L
FLUSH = 1024
STCAP = 1184
CAPB = HE + FLUSH
BK = 128

_SC_PARAMS = pltpu.CompilerParams(
    use_tc_tiling_on_sc=False, needs_layout_passes=False)


@functools.cache
def _mesh():
    return plsc.VectorSubcoreMesh(
        core_axis_name="c", subcore_axis_name="s",
        num_cores=NC, num_subcores=NS,
    )


def _part_body(src_hbm, dst_hbm, attr_hbm, pk_hbm, ab_hbm, cnt_hbm,
               si0, di0, ai0, si1, di1, ai1, pkst, atst, cntv, sm0, sm1):
    c = lax.axis_index("c")
    t = lax.axis_index("s")
    base = c * HE
    sin = (si0, si1)
    din = (di0, di1)
    ain = (ai0, ai1)
    sems = (sm0, sm1)
    ones = jnp.full((L,), 1, jnp.int32)
    zeros = jnp.full((L,), 0, jnp.int32)

    pltpu.async_copy(src_hbm.at[pl.ds(base, SEG)], si0, sm0)
    pltpu.async_copy(dst_hbm.at[pl.ds(base, SEG)], di0, sm0)
    pltpu.async_copy(attr_hbm.at[pl.ds(base, SEG)], ai0, sm0)

    def _seg_group(gr, carry):
        spv, gc = carry
        for b in range(2):
            seg = gr * 2 + b
            off = base + seg * SEG
            pltpu.make_async_copy(
                src_hbm.at[pl.ds(off, SEG)], sin[b], sems[b]).wait()
            pltpu.make_async_copy(
                dst_hbm.at[pl.ds(off, SEG)], din[b], sems[b]).wait()
            pltpu.make_async_copy(
                attr_hbm.at[pl.ds(off, SEG)], ain[b], sems[b]).wait()

            nxt = 1 - b

            @pl.when(seg + 1 < NSEG)
            def _():
                noff = base + (seg + 1) * SEG
                pltpu.async_copy(
                    src_hbm.at[pl.ds(noff, SEG)], sin[nxt], sems[nxt])
                pltpu.async_copy(
                    dst_hbm.at[pl.ds(noff, SEG)], din[nxt], sems[nxt])
                pltpu.async_copy(
                    attr_hbm.at[pl.ds(noff, SEG)], ain[nxt], sems[nxt])

            def _vgroup(j4, carry2, b=b):
                spv, gc = carry2
                for u in range(4):
                    j = j4 * 4 + u
                    d = din[b][pl.ds(j * L, L)]
                    sv = sin[b][pl.ds(j * L, L)]
                    av = ain[b][pl.ds(j * L, L)]
                    bucket = lax.shift_right_arithmetic(d * 6554, 22)
                    m = bucket == t
                    mi = jnp.where(m, ones, zeros)
                    cs = plsc.cumsum(mi)
                    tot = spv + cs
                    pos = tot - 1
                    packed = sv | lax.shift_left(d - t * RPT, 14)
                    plsc.store_scatter(pkst, [pos], packed, mask=m)
                    plsc.store_scatter(atst, [pos], av, mask=m)
                    spv = jnp.full((L,), tot[15], jnp.int32)
                fl = spv[0] >= FLUSH

                @pl.when(fl)
                def _():
                    gca = pl.multiple_of(gc, FLUSH)
                    pltpu.sync_copy(pkst.at[pl.ds(0, FLUSH)],
                                    pk_hbm.at[c, t, pl.ds(gca, FLUSH)])
                    pltpu.sync_copy(atst.at[pl.ds(0, FLUSH)],
                                    ab_hbm.at[c, t, pl.ds(gca, FLUSH)])
                    for k in range(5):
                        ks = pl.ds(k * L, L)
                        kf = pl.ds(FLUSH + k * L, L)
                        pkst[ks] = pkst[kf]
                        atst[ks] = atst[kf]

                spv = jnp.where(fl, spv - FLUSH, spv)
                gc = jnp.where(fl, gc + FLUSH, gc)
                return spv, gc

            spv, gc = lax.fori_loop(0, SEGV // 4, _vgroup, (spv, gc))
        return spv, gc

    spv, gc = lax.fori_loop(0, NSEG // 2, _seg_group,
                            (jnp.full((L,), 0, jnp.int32), jnp.int32(0)))
    sp = spv[0]

    iota = lax.broadcasted_iota(jnp.int32, (L,), 0)
    zi = jnp.zeros((L,), jnp.int32)
    zf = jnp.zeros((L,), jnp.float32)

    def _zfill(i, _):
        pos = sp + i * L + iota
        m = pos < FLUSH
        plsc.store_scatter(pkst, [pos], zi, mask=m)
        plsc.store_scatter(atst, [pos], zf, mask=m)
        return 0
    lax.fori_loop(0, FLUSH // L, _zfill, 0)
    gca = pl.multiple_of(gc, FLUSH)
    pltpu.sync_copy(pkst.at[pl.ds(0, FLUSH)],
                    pk_hbm.at[c, t, pl.ds(gca, FLUSH)])
    pltpu.sync_copy(atst.at[pl.ds(0, FLUSH)],
                    ab_hbm.at[c, t, pl.ds(gca, FLUSH)])
    gc = gc + FLUSH
    cntv[pl.ds(0, L)] = jnp.full((L,), gc, jnp.int32)
    pltpu.sync_copy(cntv, cnt_hbm.at[c, t])


@functools.cache
def _make_partition():
    return functools.partial(
        pl.kernel,
        out_type=(
            jax.ShapeDtypeStruct((NC, NS, CAPB), jnp.int32),
            jax.ShapeDtypeStruct((NC, NS, CAPB), jnp.float32),
            jax.ShapeDtypeStruct((NC, NS, L), jnp.int32),
        ),
        mesh=_mesh(),
        scratch_types=[
            pltpu.VMEM((SEG,), jnp.int32),
            pltpu.VMEM((SEG,), jnp.int32),
            pltpu.VMEM((SEG,), jnp.float32),
            pltpu.VMEM((SEG,), jnp.int32),
            pltpu.VMEM((SEG,), jnp.int32),
            pltpu.VMEM((SEG,), jnp.float32),
            pltpu.VMEM((STCAP,), jnp.int32),
            pltpu.VMEM((STCAP,), jnp.float32),
            pltpu.VMEM((L,), jnp.int32),
            pltpu.SemaphoreType.DMA,
            pltpu.SemaphoreType.DMA,
        ],
        compiler_params=_SC_PARAMS,
    )(_part_body)


SCAP = 12288


def _accum_block(fo, pkref, pkoff, abref, rbuf, acc):
    iota = lax.broadcasted_iota(jnp.int32, (L,), 0)

    def _acc(j, _):
        sl = pl.ds(pkoff + j * L, L)
        pkv = pkref[sl]
        rowbase = lax.shift_right_arithmetic(pkv, 14) * fo
        av = abref[sl]
        for h in range(2):
            sps = [jnp.full((L,), av[h * 8 + k], jnp.float32)
                   for k in range(8)]
            idxs = [jnp.full((L,), rowbase[h * 8 + k], jnp.int32) + iota
                    for k in range(8)]
            for q in range(fo // L):
                qs = pl.ds(q * L, L)
                vals = [rbuf[j * L + h * 8 + k, qs] for k in range(8)]
                prods = [vals[k] * sps[k] for k in range(8)]
                for k in range(8):
                    plsc.addupdate_scatter(acc, [idxs[k] + (q * L)],
                                           prods[k])
        return 0
    lax.fori_loop(0, BK // L, _acc, 0)


def _agg_body(fo, h_hbm, pk_hbm, ab_hbm, cnt_hbm, out_hbm,
              pks, abs_, tpk, tab, ix0, ix1, rb0, rb1, acc, cntv,
              p0, p1, g0, g1):
    ixb = (ix0, ix1)
    rbb = (rb0, rb1)
    gsem = (g0, g1)
    c = lax.axis_index("c")
    t = lax.axis_index("s")
    mask14 = jnp.full((L,), 0x3FFF, jnp.int32)

    pltpu.sync_copy(cnt_hbm.at[c, t], cntv)
    nent = cntv[pl.ds(0, L)][0]
    nb = lax.shift_right_logical(nent, 7)
    nsb = jnp.minimum(nb, SCAP // BK)

    pltpu.async_copy(pk_hbm.at[c, t, pl.ds(0, SCAP)], pks, p0)
    pltpu.async_copy(ab_hbm.at[c, t, pl.ds(0, SCAP)], abs_, p0)

    zv = jnp.zeros((L,), jnp.float32)

    def _z(r, _):
        for u in range(4):
            acc[pl.ds((r * 4 + u) * L, L)] = zv
        return 0
    lax.fori_loop(0, RPT * fo // (4 * L), _z, 0)
    pltpu.make_async_copy(pk_hbm.at[c, t, pl.ds(0, SCAP)], pks, p0).wait()
    pltpu.make_async_copy(ab_hbm.at[c, t, pl.ds(0, SCAP)], abs_, p0).wait()

    def _slot(i, b):
        @pl.when(i < nsb)
        def _():
            for j in range(BK // L):
                sl = pl.ds(j * L, L)
                ixb[b][sl] = pks[pl.ds(i * BK + j * L, L)] & mask14
            pltpu.async_copy(h_hbm.at[ixb[b]], rbb[b], gsem[b])

        @pl.when(jnp.logical_and(i >= 1, i <= nsb))
        def _():
            ob = 1 - b
            pltpu.make_async_copy(h_hbm.at[ixb[ob]], rbb[ob], gsem[ob]).wait()
            _accum_block(fo, pks, (i - 1) * BK, abs_, rbb[ob], acc)

    def _group(gi, _):
        for b in range(2):
            _slot(gi * 2 + b, b)
        return 0
    ngr = lax.shift_right_logical(nsb + 2, 1)
    lax.fori_loop(0, ngr, _group, 0)

    def _tail(i, _):
        off = pl.multiple_of(i * BK, BK)
        pltpu.sync_copy(pk_hbm.at[c, t, pl.ds(off, BK)], tpk)
        pltpu.sync_copy(ab_hbm.at[c, t, pl.ds(off, BK)], tab)
        for j in range(BK // L):
            sl = pl.ds(j * L, L)
            ixb[0][sl] = tpk[sl] & mask14
        pltpu.async_copy(h_hbm.at[ixb[0]], rbb[0], gsem[0]).wait()
        _accum_block(fo, tpk, 0, tab, rbb[0], acc)
        return 0
    lax.fori_loop(nsb, nb, _tail, 0)

    pltpu.sync_copy(acc, out_hbm.at[c, pl.ds(t * (RPT * fo), RPT * fo)])


@functools.cache
def _make_agg(fo):
    return functools.partial(
        pl.kernel,
        out_type=jax.ShapeDtypeStruct((NC, NP * fo), jnp.float32),
        mesh=_mesh(),
        scratch_types=[
            pltpu.VMEM((SCAP,), jnp.int32),
            pltpu.VMEM((SCAP,), jnp.float32),
            pltpu.VMEM((BK,), jnp.int32),
            pltpu.VMEM((BK,), jnp.float32),
            pltpu.VMEM((BK,), jnp.int32),
            pltpu.VMEM((BK,), jnp.int32),
            pltpu.VMEM((BK, fo), jnp.float32),
            pltpu.VMEM((BK, fo), jnp.float32),
            pltpu.VMEM((RPT * fo,), jnp.float32),
            pltpu.VMEM((L,), jnp.int32),
            pltpu.SemaphoreType.DMA,
            pltpu.SemaphoreType.DMA,
            pltpu.SemaphoreType.DMA,
            pltpu.SemaphoreType.DMA,
        ],
        compiler_params=_SC_PARAMS,
    )(functools.partial(_agg_body, fo))


_BR = 1024


def _mm1_body(x_ref, w_ref, o_ref):
    o_ref[...] = jnp.dot(x_ref[...], w_ref[...],
                         preferred_element_type=jnp.float32)


def _mm1(x, W):
    fi, fo = W.shape
    return pl.pallas_call(
        _mm1_body,
        grid=(NP // _BR,),
        in_specs=[pl.BlockSpec((_BR, fi), lambda i: (i, 0)),
                  pl.BlockSpec((fi, fo), lambda i: (0, 0))],
        out_specs=pl.BlockSpec((_BR, fo), lambda i: (i, 0)),
        out_shape=jax.ShapeDtypeStruct((NP, fo), jnp.float32),
    )(x, W)


def _fused_body(p_ref, b_ref, w_ref, o_ref):
    h = jnp.maximum(p_ref[0] + p_ref[1] + b_ref[...], 0.0)
    o_ref[...] = jnp.dot(h, w_ref[...], preferred_element_type=jnp.float32)


def _fused(p, b, W):
    fi, fo = W.shape
    return pl.pallas_call(
        _fused_body,
        grid=(NP // _BR,),
        in_specs=[pl.BlockSpec((2, _BR, fi), lambda i: (0, i, 0)),
                  pl.BlockSpec((1, fi), lambda i: (0, 0)),
                  pl.BlockSpec((fi, fo), lambda i: (0, 0))],
        out_specs=pl.BlockSpec((_BR, fo), lambda i: (i, 0)),
        out_shape=jax.ShapeDtypeStruct((NP, fo), jnp.float32),
    )(p, b.reshape(1, fi), W)


def _final_body(p_ref, b_ref, o_ref):
    z = p_ref[0, :, 0:1] + p_ref[1, :, 0:1] + b_ref[0, 0]
    o_ref[...] = jax.nn.sigmoid(z)


def _final(p, b):
    return pl.pallas_call(
        _final_body,
        grid=(NP // _BR,),
        in_specs=[pl.BlockSpec((2, _BR, 16), lambda i: (0, i, 0)),
                  pl.BlockSpec((1, 1), lambda i: (0, 0))],
        out_specs=pl.BlockSpec((_BR, 1), lambda i: (i, 0)),
        out_shape=jax.ShapeDtypeStruct((NP, 1), jnp.float32),
    )(p, b.reshape(1, 1))


def kernel(x, edge_index, edge_attr, W1, b1, W2, b2, W3, b3, W4, b4):
    x_p = jnp.zeros((NP, x.shape[1]), jnp.float32).at[:N].set(x)
    pad = EP - E
    src_p = jnp.concatenate([edge_index[0], jnp.zeros((pad,), jnp.int32)])
    dst_p = jnp.concatenate([edge_index[1], jnp.zeros((pad,), jnp.int32)])
    attr_p = jnp.concatenate([edge_attr, jnp.zeros((pad,), jnp.float32)])
    W4p = jnp.zeros((16, 16), jnp.float32).at[:, 0:1].set(W4)

    pk, ab, cnt = _make_partition()(src_p, dst_p, attr_p)

    h1 = _mm1(x_p, W1)
    p1 = _make_agg(64)(h1, pk, ab, cnt).reshape(NC, NP, 64)
    h2 = _fused(p1, b1, W2)
    p2 = _make_agg(32)(h2, pk, ab, cnt).reshape(NC, NP, 32)
    h3 = _fused(p2, b2, W3)
    p3 = _make_agg(16)(h3, pk, ab, cnt).reshape(NC, NP, 16)
    h4 = _fused(p3, b3, W4p)
    p4 = _make_agg(16)(h4, pk, ab, cnt).reshape(NC, NP, 16)
    out = _final(p4, b4)
    return out[:N]


# speedup vs baseline: 3.7511x; 3.4712x over previous
import functools

import jax
import jax.numpy as jnp
from jax import lax
from jax.experimental import pallas as pl
from jax.experimental.pallas import tpu as pltpu
from jax.experimental.pallas import tpu_sc as plsc

N = 10000
NP = 10240
E = 320000
NC, NS, L = 2, 16, 16
NW = NC * NS
C = 128
K = 80
EW = C * K
EP = EW * NW
RPS = NP // NS
NB = 4
D = NB - 1


@functools.cache
def _mesh():
    return plsc.VectorSubcoreMesh(
        core_axis_name="c", subcore_axis_name="s",
        num_cores=NC, num_subcores=NS,
    )


def _sc_agg_body(fo, h_hbm, src_hbm, dst_hbm, attr_hbm, out_hbm,
                 src_v, dst_v, attr_v, b0, b1, b2, b3, acc_sh, h_sh,
                 g0, g1, g2, g3, s0, s1, s2, s3):
    bufs = (b0, b1, b2, b3)
    gsem = (g0, g1, g2, g3)
    ssem = (s0, s1, s2, s3)
    c = lax.axis_index("c")
    s = lax.axis_index("s")
    w = s * NC + c

    pltpu.sync_copy(src_hbm.at[w], src_v)
    pltpu.sync_copy(dst_hbm.at[w], dst_v)
    pltpu.sync_copy(attr_hbm.at[w], attr_v)
    if fo != 64:
        hs = pl.ds(s * RPS, RPS)
        pltpu.sync_copy(h_hbm.at[hs], h_sh.at[hs])
        h_src = h_sh
    else:
        h_src = h_hbm

    if fo == 1:
        def _zero(j, _):
            b3[pl.ds(j * L, L)] = jnp.zeros((L,), jnp.float32)
            return 0
        lax.fori_loop(0, C // L, _zero, 0)
    else:
        def _zero(e, _):
            for q in range(fo // L):
                b3[e, pl.ds(q * L, L)] = jnp.zeros((L,), jnp.float32)
            return 0
        lax.fori_loop(0, C, _zero, 0)
    for r in range(RPS // C):
        pltpu.async_copy(b3, acc_sh.at[pl.ds(s * RPS + r * C, C)], g3)
    for r in range(RPS // C):
        pltpu.make_async_copy(b3, acc_sh.at[pl.ds(s * RPS + r * C, C)], g3).wait()

    plsc.subcore_barrier()
    for g in range(D):
        pltpu.async_copy(h_src.at[src_v.at[g]], bufs[g], gsem[g])

    def _group(grp, _):
        G = grp * NB
        for b in range(NB):
            g = G + b
            pb = (b - 1) % NB

            @pl.when(jnp.logical_and(g >= 1, g + D < K))
            def _():
                pltpu.make_async_copy(
                    bufs[pb], acc_sh.at[dst_v.at[g - 1]], ssem[pb]).wait()

            @pl.when(g + D < K)
            def _():
                pltpu.async_copy(
                    h_src.at[src_v.at[g + D]], bufs[pb], gsem[pb])

            pltpu.make_async_copy(
                h_src.at[src_v.at[g]], bufs[b], gsem[b]).wait()

            if fo == 1:
                for j in range(C // L):
                    sl = pl.ds(j * L, L)
                    bufs[b][sl] = bufs[b][sl] * attr_v[g, sl]
            else:
                def _mul(j, _, b=b, g=g):
                    ablk = attr_v[g, pl.ds(j * L, L)]
                    for t in range(L):
                        e = j * L + t
                        sp = jnp.full((L,), ablk[t], jnp.float32)
                        for q in range(fo // L):
                            sl = pl.ds(q * L, L)
                            bufs[b][e, sl] = bufs[b][e, sl] * sp
                    return 0
                lax.fori_loop(0, C // L, _mul, 0)

            pltpu.async_copy(
                bufs[b], acc_sh.at[dst_v.at[g]], ssem[b], add=True)
        return 0
    lax.fori_loop(0, K // NB, _group, 0)

    for b in range(NB):
        gl = K - NB + b
        pltpu.make_async_copy(bufs[b], acc_sh.at[dst_v.at[gl]], ssem[b]).wait()
    plsc.subcore_barrier()

    pltpu.sync_copy(acc_sh.at[pl.ds(s * RPS, RPS)],
                    out_hbm.at[c, pl.ds(s * RPS, RPS)])


def _sc_scratch(fo):
    if fo == 1:
        buf = lambda: pltpu.VMEM((C,), jnp.float32)
        acc = pltpu.VMEM_SHARED((NP,), jnp.float32)
        hsh = pltpu.VMEM_SHARED((NP,), jnp.float32)
    else:
        buf = lambda: pltpu.VMEM((C, fo), jnp.float32)
        acc = pltpu.VMEM_SHARED((NP, fo), jnp.float32)
        hsh = (pltpu.VMEM_SHARED((NP, fo), jnp.float32) if fo != 64
               else pltpu.VMEM_SHARED((L, L), jnp.float32))
    return [
        pltpu.VMEM((K, C), jnp.int32),
        pltpu.VMEM((K, C), jnp.int32),
        pltpu.VMEM((K, C), jnp.float32),
        buf(), buf(), buf(), buf(),
        acc, hsh,
    ] + [pltpu.SemaphoreType.DMA] * (2 * NB)


@functools.cache
def _make_sc_agg(fo):
    out_shape = (NC, NP) if fo == 1 else (NC, NP, fo)
    return functools.partial(
        pl.kernel,
        out_type=jax.ShapeDtypeStruct(out_shape, jnp.float32),
        mesh=_mesh(),
        scratch_types=_sc_scratch(fo),
        compiler_params=pltpu.CompilerParams(use_tc_tiling_on_sc=False),
    )(functools.partial(_sc_agg_body, fo))


_BR = 1024


def _mm1_body(x_ref, w_ref, o_ref):
    o_ref[...] = jnp.dot(x_ref[...], w_ref[...],
                         preferred_element_type=jnp.float32)


def _mm1(x, W):
    fi, fo = W.shape
    return pl.pallas_call(
        _mm1_body,
        grid=(NP // _BR,),
        in_specs=[pl.BlockSpec((_BR, fi), lambda i: (i, 0)),
                  pl.BlockSpec((fi, fo), lambda i: (0, 0))],
        out_specs=pl.BlockSpec((_BR, fo), lambda i: (i, 0)),
        out_shape=jax.ShapeDtypeStruct((NP, fo), jnp.float32),
    )(x, W)


def _fused_body(p_ref, b_ref, w_ref, o_ref):
    h = jnp.maximum(p_ref[0] + p_ref[1] + b_ref[...], 0.0)
    o_ref[...] = jnp.dot(h, w_ref[...], preferred_element_type=jnp.float32)


def _fused(p, b, W):
    fi, fo = W.shape
    return pl.pallas_call(
        _fused_body,
        grid=(NP // _BR,),
        in_specs=[pl.BlockSpec((2, _BR, fi), lambda i: (0, i, 0)),
                  pl.BlockSpec((1, fi), lambda i: (0, 0)),
                  pl.BlockSpec((fi, fo), lambda i: (0, 0))],
        out_specs=pl.BlockSpec((_BR, fo), lambda i: (i, 0)),
        out_shape=jax.ShapeDtypeStruct((NP, fo), jnp.float32),
    )(p, b.reshape(1, fi), W)


def _last_body(p_ref, b_ref, w_ref, o_ref):
    h = jnp.maximum(p_ref[0] + p_ref[1] + b_ref[...], 0.0)
    o_ref[...] = jnp.sum(h * w_ref[...], axis=1, keepdims=True)


def _last(p, b, W):
    fi = W.shape[0]
    return pl.pallas_call(
        _last_body,
        grid=(NP // _BR,),
        in_specs=[pl.BlockSpec((2, _BR, fi), lambda i: (0, i, 0)),
                  pl.BlockSpec((1, fi), lambda i: (0, 0)),
                  pl.BlockSpec((1, fi), lambda i: (0, 0))],
        out_specs=pl.BlockSpec((_BR, 1), lambda i: (i, 0)),
        out_shape=jax.ShapeDtypeStruct((NP, 1), jnp.float32),
    )(p, b.reshape(1, fi), W.reshape(1, fi))


def _final_body(p_ref, b_ref, o_ref):
    z = p_ref[0] + p_ref[1] + b_ref[0, 0]
    o_ref[...] = jax.nn.sigmoid(z)


def _final(p, b):
    BC = 2048
    return pl.pallas_call(
        _final_body,
        grid=(NP // BC,),
        in_specs=[pl.BlockSpec((2, BC), lambda i: (0, i)),
                  pl.BlockSpec((1, 1), lambda i: (0, 0))],
        out_specs=pl.BlockSpec((BC,), lambda i: (i,)),
        out_shape=jax.ShapeDtypeStruct((NP,), jnp.float32),
    )(p, b.reshape(1, 1))


def kernel(x, edge_index, edge_attr, W1, b1, W2, b2, W3, b3, W4, b4):
    x_p = jnp.zeros((NP, x.shape[1]), jnp.float32).at[:N].set(x)
    pad = EP - E
    src_p = jnp.concatenate(
        [edge_index[0], jnp.zeros((pad,), jnp.int32)]).reshape(NW, K, C)
    dst_p = jnp.concatenate(
        [edge_index[1], jnp.zeros((pad,), jnp.int32)]).reshape(NW, K, C)
    attr_p = jnp.concatenate(
        [edge_attr, jnp.zeros((pad,), jnp.float32)]).reshape(NW, K, C)

    h1 = _mm1(x_p, W1)
    p1 = _make_sc_agg(64)(h1, src_p, dst_p, attr_p)
    h2 = _fused(p1, b1, W2)
    p2 = _make_sc_agg(32)(h2, src_p, dst_p, attr_p)
    h3 = _fused(p2, b2, W3)
    p3 = _make_sc_agg(16)(h3, src_p, dst_p, attr_p)
    h4 = _last(p3, b3, W4)
    p4 = _make_sc_agg(1)(h4.reshape(NP), src_p, dst_p, attr_p)
    out = _final(p4, b4)
    return out[:N, None]


# speedup vs baseline: 5.6036x; 1.4939x over previous
import functools

import jax
import jax.numpy as jnp
from jax import lax
from jax.experimental import pallas as pl
from jax.experimental.pallas import tpu as pltpu
from jax.experimental.pallas import tpu_sc as plsc

N = 10000
NP = 10240
E = 320000
NC, NS, L = 2, 16, 16
NW = NC * NS
C = 128
K = 80
EW = C * K
EP = EW * NW
RPS = NP // NS
NB = 4
D = NB - 1


@functools.cache
def _mesh():
    return plsc.VectorSubcoreMesh(
        core_axis_name="c", subcore_axis_name="s",
        num_cores=NC, num_subcores=NS,
    )


def _sc_agg_body(fo, h_hbm, src_hbm, dst_hbm, attr_hbm, out_hbm,
                 src_v, dst_v, attr_v, b0, b1, b2, b3, acc_sh, h_sh,
                 g0, g1, g2, g3, s0, s1, s2, s3):
    bufs = (b0, b1, b2, b3)
    gsem = (g0, g1, g2, g3)
    ssem = (s0, s1, s2, s3)
    c = lax.axis_index("c")
    s = lax.axis_index("s")
    w = s * NC + c

    pltpu.sync_copy(src_hbm.at[w], src_v)
    pltpu.sync_copy(dst_hbm.at[w], dst_v)
    pltpu.sync_copy(attr_hbm.at[w], attr_v)
    if fo != 64:
        hs = pl.ds(s * RPS, RPS)
        pltpu.sync_copy(h_hbm.at[hs], h_sh.at[hs])
        h_src = h_sh
    else:
        h_src = h_hbm

    if fo == 1:
        def _zero(j, _):
            b3[pl.ds(j * L, L)] = jnp.zeros((L,), jnp.float32)
            return 0
        lax.fori_loop(0, C // L, _zero, 0)
    else:
        def _zero(e, _):
            for q in range(fo // L):
                b3[e, pl.ds(q * L, L)] = jnp.zeros((L,), jnp.float32)
            return 0
        lax.fori_loop(0, C, _zero, 0)
    for r in range(RPS // C):
        pltpu.async_copy(b3, acc_sh.at[pl.ds(s * RPS + r * C, C)], g3)
    for r in range(RPS // C):
        pltpu.make_async_copy(b3, acc_sh.at[pl.ds(s * RPS + r * C, C)], g3).wait()

    plsc.subcore_barrier()
    for g in range(D):
        pltpu.async_copy(h_src.at[src_v.at[g]], bufs[g], gsem[g])

    def _group(grp, _):
        G = grp * NB
        for b in range(NB):
            g = G + b
            pb = (b - 1) % NB

            @pl.when(jnp.logical_and(g >= 1, g + D < K))
            def _():
                pltpu.make_async_copy(
                    bufs[pb], acc_sh.at[dst_v.at[g - 1]], ssem[pb]).wait()

            @pl.when(g + D < K)
            def _():
                pltpu.async_copy(
                    h_src.at[src_v.at[g + D]], bufs[pb], gsem[pb])

            pltpu.make_async_copy(
                h_src.at[src_v.at[g]], bufs[b], gsem[b]).wait()

            if fo == 1:
                for j in range(C // L):
                    sl = pl.ds(j * L, L)
                    bufs[b][sl] = bufs[b][sl] * attr_v[g, sl]
            else:
                def _mul(j, _, b=b, g=g):
                    ablk = attr_v[g, pl.ds(j * L, L)]
                    for t in range(L):
                        e = j * L + t
                        sp = jnp.full((L,), ablk[t], jnp.float32)
                        for q in range(fo // L):
                            sl = pl.ds(q * L, L)
                            bufs[b][e, sl] = bufs[b][e, sl] * sp
                    return 0
                lax.fori_loop(0, C // L, _mul, 0)

            pltpu.async_copy(
                bufs[b], acc_sh.at[dst_v.at[g]], ssem[b], add=True)
        return 0
    lax.fori_loop(0, K // NB, _group, 0)

    for b in range(NB):
        gl = K - NB + b
        pltpu.make_async_copy(bufs[b], acc_sh.at[dst_v.at[gl]], ssem[b]).wait()
    plsc.subcore_barrier()

    pltpu.sync_copy(acc_sh.at[pl.ds(s * RPS, RPS)],
                    out_hbm.at[c, pl.ds(s * RPS, RPS)])


def _sc_scratch(fo):
    if fo == 1:
        buf = lambda: pltpu.VMEM((C,), jnp.float32)
        acc = pltpu.VMEM_SHARED((NP,), jnp.float32)
        hsh = pltpu.VMEM_SHARED((NP,), jnp.float32)
    else:
        buf = lambda: pltpu.VMEM((C, fo), jnp.float32)
        acc = pltpu.VMEM_SHARED((NP, fo), jnp.float32)
        hsh = (pltpu.VMEM_SHARED((NP, fo), jnp.float32) if fo != 64
               else pltpu.VMEM_SHARED((L, L), jnp.float32))
    return [
        pltpu.VMEM((K, C), jnp.int32),
        pltpu.VMEM((K, C), jnp.int32),
        pltpu.VMEM((K, C), jnp.float32),
        buf(), buf(), buf(), buf(),
        acc, hsh,
    ] + [pltpu.SemaphoreType.DMA] * (2 * NB)


@functools.cache
def _make_sc_agg(fo):
    out_shape = (NC, NP) if fo == 1 else (NC, NP, fo)
    return functools.partial(
        pl.kernel,
        out_type=jax.ShapeDtypeStruct(out_shape, jnp.float32),
        mesh=_mesh(),
        scratch_types=_sc_scratch(fo),
        compiler_params=pltpu.CompilerParams(use_tc_tiling_on_sc=False),
    )(functools.partial(_sc_agg_body, fo))


_BR = 1024


def _mm1_body(x_ref, w_ref, o_ref):
    o_ref[...] = jnp.dot(x_ref[...], w_ref[...],
                         preferred_element_type=jnp.float32)


def _mm1(x, W):
    fi, fo = W.shape
    return pl.pallas_call(
        _mm1_body,
        grid=(NP // _BR,),
        in_specs=[pl.BlockSpec((_BR, fi), lambda i: (i, 0)),
                  pl.BlockSpec((fi, fo), lambda i: (0, 0))],
        out_specs=pl.BlockSpec((_BR, fo), lambda i: (i, 0)),
        out_shape=jax.ShapeDtypeStruct((NP, fo), jnp.float32),
    )(x, W)


def _fused_body(p_ref, b_ref, w_ref, o_ref):
    h = jnp.maximum(p_ref[0] + p_ref[1] + b_ref[...], 0.0)
    o_ref[...] = jnp.dot(h, w_ref[...], preferred_element_type=jnp.float32)


def _fused(p, b, W):
    fi, fo = W.shape
    return pl.pallas_call(
        _fused_body,
        grid=(NP // _BR,),
        in_specs=[pl.BlockSpec((2, _BR, fi), lambda i: (0, i, 0)),
                  pl.BlockSpec((1, fi), lambda i: (0, 0)),
                  pl.BlockSpec((fi, fo), lambda i: (0, 0))],
        out_specs=pl.BlockSpec((_BR, fo), lambda i: (i, 0)),
        out_shape=jax.ShapeDtypeStruct((NP, fo), jnp.float32),
    )(p, b.reshape(1, fi), W)


def _last_body(p_ref, b_ref, w_ref, o_ref):
    h = jnp.maximum(p_ref[0] + p_ref[1] + b_ref[...], 0.0)
    o_ref[...] = jnp.sum(h * w_ref[...], axis=1, keepdims=True)


def _last(p, b, W):
    fi = W.shape[0]
    return pl.pallas_call(
        _last_body,
        grid=(NP // _BR,),
        in_specs=[pl.BlockSpec((2, _BR, fi), lambda i: (0, i, 0)),
                  pl.BlockSpec((1, fi), lambda i: (0, 0)),
                  pl.BlockSpec((1, fi), lambda i: (0, 0))],
        out_specs=pl.BlockSpec((_BR, 1), lambda i: (i, 0)),
        out_shape=jax.ShapeDtypeStruct((NP, 1), jnp.float32),
    )(p, b.reshape(1, fi), W.reshape(1, fi))


def _final_body(p_ref, b_ref, o_ref):
    z = p_ref[0] + p_ref[1] + b_ref[0, 0]
    o_ref[...] = jax.nn.sigmoid(z)


def _final(p, b):
    BC = 2048
    return pl.pallas_call(
        _final_body,
        grid=(NP // BC,),
        in_specs=[pl.BlockSpec((2, BC), lambda i: (0, i)),
                  pl.BlockSpec((1, 1), lambda i: (0, 0))],
        out_specs=pl.BlockSpec((BC,), lambda i: (i,)),
        out_shape=jax.ShapeDtypeStruct((NP,), jnp.float32),
    )(p, b.reshape(1, 1))


def kernel(x, edge_index, edge_attr, W1, b1, W2, b2, W3, b3, W4, b4):
    x_p = jnp.zeros((NP, x.shape[1]), jnp.float32).at[:N].set(x)
    pad = EP - E
    src_p = jnp.concatenate(
        [edge_index[0], jnp.zeros((pad,), jnp.int32)]).reshape(NW, K, C)
    dst_p = jnp.concatenate(
        [edge_index[1], jnp.zeros((pad,), jnp.int32)]).reshape(NW, K, C)
    attr_p = jnp.concatenate(
        [edge_attr, jnp.zeros((pad,), jnp.float32)]).reshape(NW, K, C)

    h1 = _mm1(x_p, W1)
    pA = _make_sc_agg(32)(h1[:, :32], src_p, dst_p, attr_p)
    pB = _make_sc_agg(32)(h1[:, 32:], src_p, dst_p, attr_p)
    p1 = jnp.concatenate([pA, pB], axis=2)
    h2 = _fused(p1, b1, W2)
    p2 = _make_sc_agg(32)(h2, src_p, dst_p, attr_p)
    h3 = _fused(p2, b2, W3)
    p3 = _make_sc_agg(16)(h3, src_p, dst_p, attr_p)
    h4 = _last(p3, b3, W4)
    p4 = _make_sc_agg(1)(h4.reshape(NP), src_p, dst_p, attr_p)
    out = _final(p4, b4)
    return out[:N, None]
